# baseline (device time: 347167 ns/iter reference)
import jax
import jax.numpy as jnp
from jax import lax
from jax.experimental import pallas as pl
from jax.experimental.pallas import tpu as pltpu

M, N = 16384, 1024
HALF = M // 2
CHUNK = 512
NC = HALF // CHUNK
NSLOT = 4


def kernel(x):
    def body(x_hbm, out_ref, recv_half, sbuf, stage_d, stage_o, obuf,
             stage_d_sems, stage_o_sems, load_sems, st_d_sems, st_o_sems,
             y_send_sems, y_recv_sems, x_send_sems, x_recv_sems):
        mx = lax.axis_index("x")
        my = lax.axis_index("y")
        ypeer = (mx, 1 - my)
        xpeer = (1 - mx, my)
        h_base = mx * HALF
        o_base = (1 - mx) * HALF

        def drow(c):
            return pl.ds(h_base + c * CHUNK, CHUNK)

        def orow(c):
            return pl.ds(o_base + c * CHUNK, CHUNK)

        def rrow(c):
            return pl.ds(c * CHUNK, CHUNK)

        barrier = pltpu.get_barrier_semaphore()
        for nbr in (ypeer, xpeer):
            pl.semaphore_signal(barrier, inc=1, device_id=nbr,
                                device_id_type=pl.DeviceIdType.MESH)
        pl.semaphore_wait(barrier, 2)

        def start_load(base, c, stage, sems, slot):
            cp = pltpu.make_async_copy(
                x_hbm.at[pl.ds(base + c * CHUNK, CHUNK), :],
                stage.at[slot],
                sems.at[slot],
            )
            cp.start()
            return cp

        y_rdmas = [None] * NC
        x_rdmas = [None] * NC
        cps = [None] * NC
        cps[0] = start_load(h_base, 0, stage_d, stage_d_sems, 0)
        if NC > 1:
            cps[1] = start_load(h_base, 1, stage_d, stage_d_sems, 1)
        for c in range(NC):
            cps[c].wait()
            if c >= NSLOT:
                y_rdmas[c - NSLOT].wait_send()
            sbuf[c % NSLOT] = stage_d[c % 2].astype(jnp.bfloat16)
            if c + 2 < NC:
                cps[c + 2] = start_load(h_base, c + 2, stage_d,
                                        stage_d_sems, c % 2)
            y_rdmas[c] = pltpu.make_async_remote_copy(
                src_ref=sbuf.at[c % NSLOT],
                dst_ref=recv_half.at[rrow(c)],
                send_sem=y_send_sems.at[c],
                recv_sem=y_recv_sems.at[c],
                device_id=ypeer,
                device_id_type=pl.DeviceIdType.MESH,
            )
            y_rdmas[c].start()

        st_d = [None] * NC
        st_o = [None] * NC
        cps_d = [None] * NC
        cps_o = [None] * NC

        cps_d[0] = start_load(h_base, 0, stage_d, stage_d_sems, 0)
        if NC > 1:
            cps_d[1] = start_load(h_base, 1, stage_d, stage_d_sems, 1)

        def finish_direct(d):
            x_rdmas[d].wait_send()
            cps_d[d].wait()
            recv_half[rrow(d), :] = (
                recv_half[rrow(d), :] + stage_d[d % 2].astype(jnp.bfloat16)
            )
            if d + 2 < NC:
                cps_d[d + 2] = start_load(h_base, d + 2, stage_d,
                                          stage_d_sems, d % 2)
            if d >= 2:
                st_d[d - 2].wait()
            st_d[d] = pltpu.make_async_copy(
                recv_half.at[rrow(d)], out_ref.at[drow(d)],
                st_d_sems.at[d % 2])
            st_d[d].start()

        def finish_other(o):
            x_rdmas[o].wait_recv()
            if o >= 2:
                st_o[o - 2].wait()
            ld = pltpu.make_async_copy(
                out_ref.at[orow(o)], obuf.at[o % 2], load_sems.at[o % 2])
            ld.start()
            ld.wait()
            cps_o[o].wait()
            obuf[o % 2] = obuf[o % 2] + stage_o[o % 2].astype(jnp.bfloat16)
            st_o[o] = pltpu.make_async_copy(
                obuf.at[o % 2], out_ref.at[orow(o)], st_o_sems.at[o % 2])
            st_o[o].start()

        for c in range(NC):
            y_rdmas[c].wait_recv()
            x_rdmas[c] = pltpu.make_async_remote_copy(
                src_ref=recv_half.at[rrow(c)],
                dst_ref=out_ref.at[drow(c)],
                send_sem=x_send_sems.at[c],
                recv_sem=x_recv_sems.at[c],
                device_id=xpeer,
                device_id_type=pl.DeviceIdType.MESH,
            )
            x_rdmas[c].start()
            if c >= 1:
                finish_direct(c - 1)
            if c >= 2:
                finish_other(c - 2)
            cps_o[c] = start_load(o_base, c, stage_o, stage_o_sems, c % 2)

        finish_direct(NC - 1)
        for o in (NC - 2, NC - 1):
            finish_other(o)
        for c in range(NC - NSLOT, NC):
            y_rdmas[c].wait_send()
        st_d[NC - 2].wait()
        st_d[NC - 1].wait()
        st_o[NC - 2].wait()
        st_o[NC - 1].wait()

    return pl.pallas_call(
        body,
        out_shape=jax.ShapeDtypeStruct((M, N), jnp.bfloat16),
        in_specs=[pl.BlockSpec(memory_space=pl.ANY)],
        out_specs=pl.BlockSpec(memory_space=pl.ANY),
        scratch_shapes=[
            pltpu.VMEM((HALF, N), jnp.bfloat16),
            pltpu.VMEM((NSLOT, CHUNK, N), jnp.bfloat16),
            pltpu.VMEM((2, CHUNK, N), jnp.float32),
            pltpu.VMEM((2, CHUNK, N), jnp.float32),
            pltpu.VMEM((2, CHUNK, N), jnp.bfloat16),
            pltpu.SemaphoreType.DMA((2,)),
            pltpu.SemaphoreType.DMA((2,)),
            pltpu.SemaphoreType.DMA((2,)),
            pltpu.SemaphoreType.DMA((2,)),
            pltpu.SemaphoreType.DMA((2,)),
            pltpu.SemaphoreType.DMA((NC,)),
            pltpu.SemaphoreType.DMA((NC,)),
            pltpu.SemaphoreType.DMA((NC,)),
            pltpu.SemaphoreType.DMA((NC,)),
        ],
        compiler_params=pltpu.CompilerParams(collective_id=0),
    )(x)


# device time: 225862 ns/iter; 1.5371x vs baseline; 1.5371x over previous
import jax
import jax.numpy as jnp
from jax import lax
from jax.experimental import pallas as pl
from jax.experimental.pallas import tpu as pltpu

M, N = 16384, 1024
HALF = M // 2
CHUNK = 512
NC = HALF // CHUNK


def kernel(x):
    def body(x_hbm, out_ref, send_half, stage_d, stage_o, obuf,
             stage_d_sems, stage_o_sems, load_sems, st_d_sems, st_o_sems,
             y_send_sems, y_recv_sems, x_send_sems, x_recv_sems):
        mx = lax.axis_index("x")
        my = lax.axis_index("y")
        ypeer = (mx, 1 - my)
        xpeer = (1 - mx, my)
        h_base = mx * HALF
        o_base = (1 - mx) * HALF

        def drow(c):
            return pl.ds(h_base + c * CHUNK, CHUNK)

        def orow(c):
            return pl.ds(o_base + c * CHUNK, CHUNK)

        barrier = pltpu.get_barrier_semaphore()
        for nbr in (ypeer, xpeer):
            pl.semaphore_signal(barrier, inc=1, device_id=nbr,
                                device_id_type=pl.DeviceIdType.MESH)
        pl.semaphore_wait(barrier, 2)

        def start_load(base, c, stage, sems, slot):
            cp = pltpu.make_async_copy(
                x_hbm.at[pl.ds(base + c * CHUNK, CHUNK), :],
                stage.at[slot],
                sems.at[slot],
            )
            cp.start()
            return cp

        y_rdmas = [None] * NC
        x_rdmas = [None] * NC
        ld_d = [None] * NC
        st_d = [None] * NC
        st_o = [None] * NC
        cps = [None] * NC
        cps_o = [None] * NC

        def cast_and_send(c):
            cps[c].wait()
            send_half[pl.ds(c * CHUNK, CHUNK), :] = (
                stage_d[c % 2].astype(jnp.bfloat16)
            )
            if c + 2 < NC:
                cps[c + 2] = start_load(h_base, c + 2, stage_d,
                                        stage_d_sems, c % 2)
            y_rdmas[c] = pltpu.make_async_remote_copy(
                src_ref=send_half.at[pl.ds(c * CHUNK, CHUNK), :],
                dst_ref=out_ref.at[drow(c)],
                send_sem=y_send_sems.at[c],
                recv_sem=y_recv_sems.at[c],
                device_id=ypeer,
                device_id_type=pl.DeviceIdType.MESH,
            )
            y_rdmas[c].start()

        def forward(f):
            y_rdmas[f].wait_recv()
            x_rdmas[f] = pltpu.make_async_remote_copy(
                src_ref=out_ref.at[drow(f)],
                dst_ref=out_ref.at[drow(f)],
                send_sem=x_send_sems.at[f],
                recv_sem=x_recv_sems.at[f],
                device_id=xpeer,
                device_id_type=pl.DeviceIdType.MESH,
            )
            x_rdmas[f].start()
            if f >= 2:
                st_d[f - 2].wait()
            ld_d[f] = pltpu.make_async_copy(
                out_ref.at[drow(f)], obuf.at[f % 2], load_sems.at[f % 2])
            ld_d[f].start()

        def finish_direct(d):
            ld_d[d].wait()
            obuf[d % 2] = obuf[d % 2] + send_half[pl.ds(d * CHUNK, CHUNK), :]
            y_rdmas[d].wait_send()
            x_rdmas[d].wait_send()
            st_d[d] = pltpu.make_async_copy(
                obuf.at[d % 2], out_ref.at[drow(d)], st_d_sems.at[d % 2])
            st_d[d].start()

        def finish_other(o):
            oslot = 2 + (o % 2)
            x_rdmas[o].wait_recv()
            if o >= 2:
                st_o[o - 2].wait()
            ld = pltpu.make_async_copy(
                out_ref.at[orow(o)], obuf.at[oslot], load_sems.at[oslot])
            ld.start()
            ld.wait()
            cps_o[o].wait()
            obuf[oslot] = obuf[oslot] + stage_o[o % 2].astype(jnp.bfloat16)
            st_o[o] = pltpu.make_async_copy(
                obuf.at[oslot], out_ref.at[orow(o)], st_o_sems.at[o % 2])
            st_o[o].start()

        cps[0] = start_load(h_base, 0, stage_d, stage_d_sems, 0)
        if NC > 1:
            cps[1] = start_load(h_base, 1, stage_d, stage_d_sems, 1)
        for c in range(NC):
            cast_and_send(c)
            if c >= 1:
                forward(c - 1)
            if c >= 2:
                finish_direct(c - 2)
            if c >= 3:
                finish_other(c - 3)
            if c >= 1:
                cps_o[c - 1] = start_load(o_base, c - 1, stage_o,
                                          stage_o_sems, (c - 1) % 2)

        forward(NC - 1)
        finish_direct(NC - 2)
        finish_other(NC - 3)
        cps_o[NC - 1] = start_load(o_base, NC - 1, stage_o,
                                   stage_o_sems, (NC - 1) % 2)
        finish_direct(NC - 1)
        finish_other(NC - 2)
        finish_other(NC - 1)
        st_d[NC - 2].wait()
        st_d[NC - 1].wait()
        st_o[NC - 2].wait()
        st_o[NC - 1].wait()

    return pl.pallas_call(
        body,
        out_shape=jax.ShapeDtypeStruct((M, N), jnp.bfloat16),
        in_specs=[pl.BlockSpec(memory_space=pl.ANY)],
        out_specs=pl.BlockSpec(memory_space=pl.ANY),
        scratch_shapes=[
            pltpu.VMEM((HALF, N), jnp.bfloat16),
            pltpu.VMEM((2, CHUNK, N), jnp.float32),
            pltpu.VMEM((2, CHUNK, N), jnp.float32),
            pltpu.VMEM((4, CHUNK, N), jnp.bfloat16),
            pltpu.SemaphoreType.DMA((2,)),
            pltpu.SemaphoreType.DMA((2,)),
            pltpu.SemaphoreType.DMA((4,)),
            pltpu.SemaphoreType.DMA((2,)),
            pltpu.SemaphoreType.DMA((2,)),
            pltpu.SemaphoreType.DMA((NC,)),
            pltpu.SemaphoreType.DMA((NC,)),
            pltpu.SemaphoreType.DMA((NC,)),
            pltpu.SemaphoreType.DMA((NC,)),
        ],
        compiler_params=pltpu.CompilerParams(collective_id=0),
    )(x)
